# baseline (speedup 1.0000x reference)
import functools

import jax
import jax.numpy as jnp
from jax import lax
from jax.experimental import pallas as pl
from jax.experimental.pallas import tpu as pltpu
from jax.experimental.pallas import tpu_sc as plsc

_N = 4194304
_NC = 2
_NS = 16
_NW = _NC * _NS
_PW = _N // _NW
_CH = 16384
_NCH = _PW // _CH
_BLK = 1024
_SPB = _BLK // 16
_BPC = _CH // _BLK
_NBLK = _PW // _BLK

_NEG = -3.4e38
_BIGI = 2**30


def _lane_max(vec):
    idx = lax.iota(jnp.int32, 16)
    dnums = lax.GatherDimensionNumbers(
        offset_dims=(), collapsed_slice_dims=(0,), start_index_map=(0,))
    for sh in (8, 4, 2, 1):
        perm = jnp.bitwise_xor(idx, sh)
        shuf = lax.gather(vec, perm[:, None], dnums, slice_sizes=(1,),
                          unique_indices=True, indices_are_sorted=False,
                          mode=lax.GatherScatterMode.PROMISE_IN_BOUNDS)
        vec = jnp.maximum(vec, shuf)
    return vec[0]


def _scan_body(in_hbm, tg_hbm, v_out, i_out, a_out, b_out,
               in_a, in_b, tg_a, tg_b, bmax,
               vscr, iscr, ascr, bscr, sem_a, sem_b):
    cid = lax.axis_index("c")
    sid = lax.axis_index("s")
    wid = sid * _NC + cid
    base = wid * _PW

    in_bufs = (in_a, in_b)
    tg_bufs = (tg_a, tg_b)
    sems = (sem_a, sem_b)

    def fire(c):
        par = c % 2
        cpa = pltpu.make_async_copy(
            in_hbm.at[pl.ds(base + c * _CH, _CH)], in_bufs[par], sems[par])
        cpb = pltpu.make_async_copy(
            tg_hbm.at[pl.ds(base + c * _CH, _CH)], tg_bufs[par], sems[par])
        cpa.start()
        cpb.start()
        return cpa, cpb

    pend = fire(0)
    vglob = jnp.zeros((16,), jnp.float32)
    for c in range(_NCH):
        nxt = fire(c + 1) if c + 1 < _NCH else None
        pend[0].wait()
        pend[1].wait()
        pend = nxt
        ibuf = in_bufs[c % 2]
        tbuf = tg_bufs[c % 2]

        def blk_body(bi, vg, ibuf=ibuf, tbuf=tbuf, c=c):
            def s_body(si, vmax):
                off = bi * _BLK + si * 16
                a = ibuf[pl.ds(off, 16)]
                t = tbuf[pl.ds(off, 16)]
                d = a - t
                return jnp.maximum(vmax, d * d)

            vmax = lax.fori_loop(0, _SPB, s_body, jnp.zeros((16,), jnp.float32))
            bmax[pl.ds((c * _BPC + bi) * 16, 16)] = vmax
            return jnp.maximum(vg, vmax)

        vglob = lax.fori_loop(0, _BPC, blk_body, vglob)

    m_val = _lane_max(vglob)

    def red_body(b, b_cur):
        vec = bmax[pl.ds(b * 16, 16)]
        hit = _lane_max(jnp.where(vec == m_val, 1.0, 0.0)) > 0.5
        return jnp.where(hit & (b_cur == _NBLK), b, b_cur)

    b_star = lax.fori_loop(0, _NBLK, red_body, jnp.int32(_NBLK))
    b_star = jnp.minimum(b_star, _NBLK - 1)

    gbase = base + b_star * _BLK
    cpa = pltpu.make_async_copy(
        in_hbm.at[pl.ds(gbase, _BLK)], in_a.at[pl.ds(0, _BLK)], sem_a)
    cpb = pltpu.make_async_copy(
        tg_hbm.at[pl.ds(gbase, _BLK)], tg_a.at[pl.ds(0, _BLK)], sem_a)
    cpa.start()
    cpb.start()
    cpa.wait()
    cpb.wait()

    lane = lax.iota(jnp.int32, 16)

    def rs_body(si, carry):
        bidx, b_a, b_b = carry
        a = in_a[pl.ds(si * 16, 16)]
        t = tg_a[pl.ds(si * 16, 16)]
        d = a - t
        d2 = d * d
        idxv = gbase + si * 16 + lane
        hit = (d2 == m_val) & (idxv < bidx)
        return (jnp.where(hit, idxv, bidx),
                jnp.where(hit, a, b_a),
                jnp.where(hit, t, b_b))

    bidx, b_a, b_b = lax.fori_loop(
        0, _SPB, rs_body,
        (jnp.full((16,), _BIGI, jnp.int32),
         jnp.zeros((16,), jnp.float32),
         jnp.zeros((16,), jnp.float32)))

    found = bidx < _BIGI
    vscr[...] = jnp.where(found, m_val, _NEG)
    iscr[...] = bidx
    ascr[...] = b_a
    bscr[...] = b_b
    pltpu.sync_copy(vscr, v_out.at[pl.ds(wid * 16, 16)])
    pltpu.sync_copy(iscr, i_out.at[pl.ds(wid * 16, 16)])
    pltpu.sync_copy(ascr, a_out.at[pl.ds(wid * 16, 16)])
    pltpu.sync_copy(bscr, b_out.at[pl.ds(wid * 16, 16)])


_phase1 = pl.kernel(
    _scan_body,
    out_type=[
        jax.ShapeDtypeStruct((_NW * 16,), jnp.float32),
        jax.ShapeDtypeStruct((_NW * 16,), jnp.int32),
        jax.ShapeDtypeStruct((_NW * 16,), jnp.float32),
        jax.ShapeDtypeStruct((_NW * 16,), jnp.float32),
    ],
    mesh=plsc.VectorSubcoreMesh(
        core_axis_name="c", subcore_axis_name="s",
        num_cores=_NC, num_subcores=_NS),
    scratch_types=[
        pltpu.VMEM((_CH,), jnp.float32),
        pltpu.VMEM((_CH,), jnp.float32),
        pltpu.VMEM((_CH,), jnp.float32),
        pltpu.VMEM((_CH,), jnp.float32),
        pltpu.VMEM((_NBLK * 16,), jnp.float32),
        pltpu.VMEM((16,), jnp.float32),
        pltpu.VMEM((16,), jnp.int32),
        pltpu.VMEM((16,), jnp.float32),
        pltpu.VMEM((16,), jnp.float32),
        pltpu.SemaphoreType.DMA,
        pltpu.SemaphoreType.DMA,
    ],
)


def _merge_body(v_ref, i_ref, a_ref, b_ref, md_ref, p_ref, ac_ref):
    v = v_ref[...]
    idx = i_ref[...]
    a = a_ref[...]
    b = b_ref[...]
    m_g = jnp.max(v)
    maskv = v == m_g
    g_i = jnp.min(jnp.where(maskv, idx, _BIGI))
    sel = maskv & (idx == g_i)
    md_ref[0] = jnp.sqrt(m_g)
    p_ref[0] = jnp.max(jnp.where(sel, a, _NEG))
    ac_ref[0] = jnp.max(jnp.where(sel, b, _NEG))


_phase2 = pl.pallas_call(
    _merge_body,
    out_shape=[
        jax.ShapeDtypeStruct((1,), jnp.float32),
        jax.ShapeDtypeStruct((1,), jnp.float32),
        jax.ShapeDtypeStruct((1,), jnp.float32),
    ],
    out_specs=[
        pl.BlockSpec(memory_space=pltpu.SMEM),
        pl.BlockSpec(memory_space=pltpu.SMEM),
        pl.BlockSpec(memory_space=pltpu.SMEM),
    ],
)


def kernel(inputs, target):
    v, idx, a, b = _phase1(inputs, target)
    md, p, ac = _phase2(
        v.reshape(4, 128), idx.reshape(4, 128),
        a.reshape(4, 128), b.reshape(4, 128))
    return (md[0], p[0], ac[0])


# speedup vs baseline: 1.2872x; 1.2872x over previous
import functools

import jax
import jax.numpy as jnp
from jax import lax
from jax.experimental import pallas as pl
from jax.experimental.pallas import tpu as pltpu
from jax.experimental.pallas import tpu_sc as plsc

_N = 4194304
_NC = 2
_NS = 16
_NW = _NC * _NS
_PW = _N // _NW
_CH = 16384
_NCH = _PW // _CH
_BLK = 1024
_SPB = _BLK // 16
_BPC = _CH // _BLK
_NBLK = _PW // _BLK

_NEG = -3.4e38
_BIGI = 2**30


def _lane_max(vec):
    idx = lax.iota(jnp.int32, 16)
    dnums = lax.GatherDimensionNumbers(
        offset_dims=(), collapsed_slice_dims=(0,), start_index_map=(0,))
    for sh in (8, 4, 2, 1):
        perm = jnp.bitwise_xor(idx, sh)
        shuf = lax.gather(vec, perm[:, None], dnums, slice_sizes=(1,),
                          unique_indices=True, indices_are_sorted=False,
                          mode=lax.GatherScatterMode.PROMISE_IN_BOUNDS)
        vec = jnp.maximum(vec, shuf)
    return vec[0]


def _scan_body(in_hbm, tg_hbm, v_out, i_out, a_out, b_out,
               in_a, in_b, tg_a, tg_b, bmax,
               vscr, iscr, ascr, bscr, sem_a, sem_b):
    cid = lax.axis_index("c")
    sid = lax.axis_index("s")
    wid = sid * _NC + cid
    base = wid * _PW

    in_bufs = (in_a, in_b)
    tg_bufs = (tg_a, tg_b)
    sems = (sem_a, sem_b)

    def fire(c):
        par = c % 2
        cpa = pltpu.make_async_copy(
            in_hbm.at[pl.ds(base + c * _CH, _CH)], in_bufs[par], sems[par])
        cpb = pltpu.make_async_copy(
            tg_hbm.at[pl.ds(base + c * _CH, _CH)], tg_bufs[par], sems[par])
        cpa.start()
        cpb.start()
        return cpa, cpb

    pend = fire(0)
    vglob = jnp.zeros((16,), jnp.float32)
    for c in range(_NCH):
        nxt = fire(c + 1) if c + 1 < _NCH else None
        pend[0].wait()
        pend[1].wait()
        pend = nxt
        ibuf = in_bufs[c % 2]
        tbuf = tg_bufs[c % 2]

        @plsc.parallel_loop(0, _BPC, carry=vglob)
        def blk_body(bi, vg, ibuf=ibuf, tbuf=tbuf, c=c):
            accs = [jnp.zeros((16,), jnp.float32) for _ in range(4)]
            for s in range(_SPB):
                off = bi * _BLK + s * 16
                a = ibuf[pl.ds(off, 16)]
                t = tbuf[pl.ds(off, 16)]
                d = a - t
                accs[s % 4] = jnp.maximum(accs[s % 4], d * d)
            vmax = jnp.maximum(jnp.maximum(accs[0], accs[1]),
                               jnp.maximum(accs[2], accs[3]))
            bmax[pl.ds((c * _BPC + bi) * 16, 16)] = vmax
            return jnp.maximum(vg, vmax)

        vglob = blk_body

    m_val = _lane_max(vglob)

    def red_body(b, bf):
        vec = bmax[pl.ds(b * 16, 16)]
        hitv = (vec == m_val) & (bf == _NBLK)
        return jnp.where(hitv, b, bf)

    bfirst = lax.fori_loop(
        0, _NBLK, red_body, jnp.full((16,), _NBLK, jnp.int32))
    b_star = (-_lane_max(-bfirst.astype(jnp.float32))).astype(jnp.int32)
    b_star = jnp.minimum(b_star, _NBLK - 1)

    gbase = base + b_star * _BLK
    cpa = pltpu.make_async_copy(
        in_hbm.at[pl.ds(gbase, _BLK)], in_a.at[pl.ds(0, _BLK)], sem_a)
    cpb = pltpu.make_async_copy(
        tg_hbm.at[pl.ds(gbase, _BLK)], tg_a.at[pl.ds(0, _BLK)], sem_a)
    cpa.start()
    cpb.start()
    cpa.wait()
    cpb.wait()

    lane = lax.iota(jnp.int32, 16)

    def rs_body(si, carry):
        bidx, b_a, b_b = carry
        a = in_a[pl.ds(si * 16, 16)]
        t = tg_a[pl.ds(si * 16, 16)]
        d = a - t
        d2 = d * d
        idxv = gbase + si * 16 + lane
        hit = (d2 == m_val) & (idxv < bidx)
        return (jnp.where(hit, idxv, bidx),
                jnp.where(hit, a, b_a),
                jnp.where(hit, t, b_b))

    bidx, b_a, b_b = lax.fori_loop(
        0, _SPB, rs_body,
        (jnp.full((16,), _BIGI, jnp.int32),
         jnp.zeros((16,), jnp.float32),
         jnp.zeros((16,), jnp.float32)))

    found = bidx < _BIGI
    vscr[...] = jnp.where(found, m_val, _NEG)
    iscr[...] = bidx
    ascr[...] = b_a
    bscr[...] = b_b
    pltpu.sync_copy(vscr, v_out.at[pl.ds(wid * 16, 16)])
    pltpu.sync_copy(iscr, i_out.at[pl.ds(wid * 16, 16)])
    pltpu.sync_copy(ascr, a_out.at[pl.ds(wid * 16, 16)])
    pltpu.sync_copy(bscr, b_out.at[pl.ds(wid * 16, 16)])


_phase1 = pl.kernel(
    _scan_body,
    out_type=[
        jax.ShapeDtypeStruct((_NW * 16,), jnp.float32),
        jax.ShapeDtypeStruct((_NW * 16,), jnp.int32),
        jax.ShapeDtypeStruct((_NW * 16,), jnp.float32),
        jax.ShapeDtypeStruct((_NW * 16,), jnp.float32),
    ],
    mesh=plsc.VectorSubcoreMesh(
        core_axis_name="c", subcore_axis_name="s",
        num_cores=_NC, num_subcores=_NS),
    scratch_types=[
        pltpu.VMEM((_CH,), jnp.float32),
        pltpu.VMEM((_CH,), jnp.float32),
        pltpu.VMEM((_CH,), jnp.float32),
        pltpu.VMEM((_CH,), jnp.float32),
        pltpu.VMEM((_NBLK * 16,), jnp.float32),
        pltpu.VMEM((16,), jnp.float32),
        pltpu.VMEM((16,), jnp.int32),
        pltpu.VMEM((16,), jnp.float32),
        pltpu.VMEM((16,), jnp.float32),
        pltpu.SemaphoreType.DMA,
        pltpu.SemaphoreType.DMA,
    ],
)


def _merge_body(v_ref, i_ref, a_ref, b_ref, md_ref, p_ref, ac_ref):
    v = v_ref[...]
    idx = i_ref[...]
    a = a_ref[...]
    b = b_ref[...]
    m_g = jnp.max(v)
    maskv = v == m_g
    g_i = jnp.min(jnp.where(maskv, idx, _BIGI))
    sel = maskv & (idx == g_i)
    md_ref[0] = jnp.sqrt(m_g)
    p_ref[0] = jnp.max(jnp.where(sel, a, _NEG))
    ac_ref[0] = jnp.max(jnp.where(sel, b, _NEG))


_phase2 = pl.pallas_call(
    _merge_body,
    out_shape=[
        jax.ShapeDtypeStruct((1,), jnp.float32),
        jax.ShapeDtypeStruct((1,), jnp.float32),
        jax.ShapeDtypeStruct((1,), jnp.float32),
    ],
    out_specs=[
        pl.BlockSpec(memory_space=pltpu.SMEM),
        pl.BlockSpec(memory_space=pltpu.SMEM),
        pl.BlockSpec(memory_space=pltpu.SMEM),
    ],
)


def kernel(inputs, target):
    v, idx, a, b = _phase1(inputs, target)
    md, p, ac = _phase2(
        v.reshape(4, 128), idx.reshape(4, 128),
        a.reshape(4, 128), b.reshape(4, 128))
    return (md[0], p[0], ac[0])
